# baseline (speedup 1.0000x reference)
import functools

import jax
import jax.numpy as jnp
from jax import lax
from jax.experimental import pallas as pl
from jax.experimental.pallas import tpu as pltpu, tpu_sc as plsc

_ROWS = 16384
_COLS = 100
_TOTAL = _ROWS * _COLS
_L = 16
_NW = 32

_ROWS_SC = 4096
_ROWS_W = _ROWS_SC // _NW
_CH_ROWS = 64
_NCH = _ROWS_W // _CH_ROWS
_NBUF = 2
_FULL_VECS = _COLS // _L
_TAIL_OFF = _COLS - _L
_TAIL_NEW = _L - (_COLS - _FULL_VECS * _L)

_TC_BR = 1024
_TC_BLOCKS = (_ROWS - _ROWS_SC) // _TC_BR
_TC_OFF = _ROWS_SC // _TC_BR


def _make_sc_kernel():
    mesh = plsc.VectorSubcoreMesh(core_axis_name="c", subcore_axis_name="s")

    @functools.partial(
        pl.kernel,
        mesh=mesh,
        out_type=jax.ShapeDtypeStruct((_NW, 2, _L), jnp.int32),
        scratch_types=[
            pltpu.VMEM((_NBUF, _CH_ROWS, _COLS), jnp.float32),
            pltpu.VMEM((_NBUF, _CH_ROWS, _COLS), jnp.int32),
            pltpu.VMEM((2, _L), jnp.int32),
            pltpu.SemaphoreType.DMA,
            pltpu.SemaphoreType.DMA,
        ],
    )
    def conf(out_hbm, tgt_hbm, res_hbm, o_v, t_v, r_v, sem0, sem1):
        nc = lax.axis_index("c")
        ns = lax.axis_index("s")
        wid = ns * 2 + nc
        row0 = wid * _ROWS_W
        sems = (sem0, sem1)

        def start(buf, c):
            r = row0 + c * _CH_ROWS
            return (
                pltpu.async_copy(
                    out_hbm.at[pl.ds(r, _CH_ROWS), :], o_v.at[buf], sems[buf]
                ),
                pltpu.async_copy(
                    tgt_hbm.at[pl.ds(r, _CH_ROWS), :], t_v.at[buf], sems[buf]
                ),
            )

        zeros = jnp.zeros((_L,), jnp.int32)
        tail_ok = lax.iota(jnp.int32, _L) >= _TAIL_NEW

        handles = [None, None]
        handles[0] = start(0, 0)

        acc_ac = zeros
        acc_b = zeros
        for c in range(_NCH):
            buf = c % _NBUF
            if c + 1 < _NCH:
                handles[(c + 1) % _NBUF] = start((c + 1) % _NBUF, c + 1)
            ha, hb = handles[buf]
            ha.wait()
            hb.wait()

            def body(r, carry, buf=buf):
                a_ac, a_b = carry
                for j in range(_FULL_VECS):
                    o = o_v[buf, r, pl.ds(j * _L, _L)]
                    t = t_v[buf, r, pl.ds(j * _L, _L)]
                    p = o > 0.0
                    a_ac = a_ac + jnp.where(p, t + (1 << 16), zeros)
                    a_b = a_b + t
                o = o_v[buf, r, pl.ds(_TAIL_OFF, _L)]
                t = t_v[buf, r, pl.ds(_TAIL_OFF, _L)]
                p = jnp.logical_and(o > 0.0, tail_ok)
                a_ac = a_ac + jnp.where(p, t + (1 << 16), zeros)
                a_b = a_b + jnp.where(tail_ok, t, zeros)
                return (a_ac, a_b)

            acc_ac, acc_b = lax.fori_loop(
                0, _CH_ROWS, body, (acc_ac, acc_b), unroll=2
            )

        r_v[0, :] = acc_ac
        r_v[1, :] = acc_b
        pltpu.sync_copy(r_v, res_hbm.at[wid])

    return conf


def _tc_body(x_ref, t_ref, ac_ref, b_ref):
    @pl.when(pl.program_id(0) == 0)
    def _():
        ac_ref[...] = jnp.zeros_like(ac_ref)
        b_ref[...] = jnp.zeros_like(b_ref)

    x = x_ref[...]
    t = t_ref[...]
    contrib = jnp.where(x > 0.0, t + (1 << 16), 0)
    ac_ref[...] += contrib.reshape(_TC_BR // 8, 8, _COLS).sum(axis=0)
    b_ref[...] += t.reshape(_TC_BR // 8, 8, _COLS).sum(axis=0)


_tc_kernel = pl.pallas_call(
    _tc_body,
    grid=(_TC_BLOCKS,),
    in_specs=[
        pl.BlockSpec((_TC_BR, _COLS), lambda i: (i + _TC_OFF, 0)),
        pl.BlockSpec((_TC_BR, _COLS), lambda i: (i + _TC_OFF, 0)),
    ],
    out_specs=[
        pl.BlockSpec((8, _COLS), lambda i: (0, 0)),
        pl.BlockSpec((8, _COLS), lambda i: (0, 0)),
    ],
    out_shape=[
        jax.ShapeDtypeStruct((8, _COLS), jnp.int32),
        jax.ShapeDtypeStruct((8, _COLS), jnp.int32),
    ],
    compiler_params=pltpu.CompilerParams(
        dimension_semantics=("arbitrary",),
    ),
)

_sc_kernel = _make_sc_kernel()


def kernel(output, target):
    sc_res = _sc_kernel(output, target)
    tc_ac, tc_b = _tc_kernel(output, target)
    a = jnp.sum(sc_res[:, 0, :] >> 16) + jnp.sum(tc_ac >> 16)
    c = jnp.sum(sc_res[:, 0, :] & 0xFFFF) + jnp.sum(tc_ac & 0xFFFF)
    b = jnp.sum(sc_res[:, 1, :]) + jnp.sum(tc_b)
    tp = c
    fp = a - c
    fn = _TOTAL - b - a + c
    return (tp, fp, fn)


# speedup vs baseline: 1.0742x; 1.0742x over previous
import functools

import jax
import jax.numpy as jnp
from jax import lax
from jax.experimental import pallas as pl
from jax.experimental.pallas import tpu as pltpu, tpu_sc as plsc

_ROWS = 16384
_COLS = 100
_TOTAL = _ROWS * _COLS
_L = 16
_NW = 32
_ROWS_W = _ROWS // _NW
_CH_ROWS = 128
_NCH = _ROWS_W // _CH_ROWS
_NBUF = 2
_FULL_VECS = _COLS // _L
_TAIL_OFF = _COLS - _L
_TAIL_NEW = _L - (_COLS - _FULL_VECS * _L)


def _make_conf_kernel():
    mesh = plsc.VectorSubcoreMesh(core_axis_name="c", subcore_axis_name="s")

    @functools.partial(
        pl.kernel,
        mesh=mesh,
        out_type=jax.ShapeDtypeStruct((_NW, 2, _L), jnp.int32),
        scratch_types=[
            pltpu.VMEM((_NBUF, _CH_ROWS, _COLS), jnp.float32),
            pltpu.VMEM((_NBUF, _CH_ROWS, _COLS), jnp.int32),
            pltpu.VMEM((2, _L), jnp.int32),
            pltpu.SemaphoreType.DMA,
            pltpu.SemaphoreType.DMA,
        ],
        compiler_params=pltpu.CompilerParams(skip_device_barrier=True),
    )
    def conf(out_hbm, tgt_hbm, res_hbm, o_v, t_v, r_v, sem0, sem1):
        nc = lax.axis_index("c")
        ns = lax.axis_index("s")
        wid = ns * 2 + nc
        row0 = wid * _ROWS_W
        sems = (sem0, sem1)

        def start(buf, c):
            r = row0 + c * _CH_ROWS
            return (
                pltpu.async_copy(
                    out_hbm.at[pl.ds(r, _CH_ROWS), :], o_v.at[buf], sems[buf]
                ),
                pltpu.async_copy(
                    tgt_hbm.at[pl.ds(r, _CH_ROWS), :], t_v.at[buf], sems[buf]
                ),
            )

        zeros = jnp.zeros((_L,), jnp.int32)
        tail_ok = lax.iota(jnp.int32, _L) >= _TAIL_NEW

        handles = [None, None]
        handles[0] = start(0, 0)

        acc = [zeros, zeros, zeros, zeros]
        for c in range(_NCH):
            buf = c % _NBUF
            if c + 1 < _NCH:
                handles[(c + 1) % _NBUF] = start((c + 1) % _NBUF, c + 1)
            ha, hb = handles[buf]
            ha.wait()
            hb.wait()

            def body(r, carry, buf=buf):
                accs = list(carry)
                for j in range(_FULL_VECS):
                    o = o_v[buf, r, pl.ds(j * _L, _L)]
                    t = t_v[buf, r, pl.ds(j * _L, _L)]
                    p = o > 0.0
                    k = j % 2
                    accs[2 * k] = accs[2 * k] + jnp.where(
                        p, t + (1 << 16), zeros
                    )
                    accs[2 * k + 1] = accs[2 * k + 1] + t
                o = o_v[buf, r, pl.ds(_TAIL_OFF, _L)]
                t = t_v[buf, r, pl.ds(_TAIL_OFF, _L)]
                p = jnp.logical_and(o > 0.0, tail_ok)
                accs[0] = accs[0] + jnp.where(p, t + (1 << 16), zeros)
                accs[1] = accs[1] + jnp.where(tail_ok, t, zeros)
                return tuple(accs)

            acc = lax.fori_loop(
                0, _CH_ROWS, body, tuple(acc), unroll=4
            )

        r_v[0, :] = acc[0] + acc[2]
        r_v[1, :] = acc[1] + acc[3]
        pltpu.sync_copy(r_v, res_hbm.at[wid])

    return conf


_conf = _make_conf_kernel()


def kernel(output, target):
    res = _conf(output, target)
    a = jnp.sum(res[:, 0, :] >> 16)
    c = jnp.sum(res[:, 0, :] & 0xFFFF)
    b = jnp.sum(res[:, 1, :])
    tp = c
    fp = a - c
    fn = _TOTAL - b - a + c
    return (tp, fp, fn)


# speedup vs baseline: 1.0758x; 1.0015x over previous
import functools

import jax
import jax.numpy as jnp
from jax import lax
from jax.experimental import pallas as pl
from jax.experimental.pallas import tpu as pltpu, tpu_sc as plsc

_ROWS = 16384
_COLS = 100
_TOTAL = _ROWS * _COLS
_L = 16
_NW = 32
_ROWS_W = _ROWS // _NW
_CH_ROWS = 128
_NCH = _ROWS_W // _CH_ROWS
_NBUF = 2
_FULL_VECS = _COLS // _L
_TAIL_OFF = _COLS - _L
_TAIL_NEW = _L - (_COLS - _FULL_VECS * _L)


def _make_conf_kernel():
    mesh = plsc.VectorSubcoreMesh(core_axis_name="c", subcore_axis_name="s")

    @functools.partial(
        pl.kernel,
        mesh=mesh,
        out_type=jax.ShapeDtypeStruct((_NW, 2, _L), jnp.int32),
        scratch_types=[
            pltpu.VMEM((_NBUF, _CH_ROWS, _COLS), jnp.float32),
            pltpu.VMEM((_NBUF, _CH_ROWS, _COLS), jnp.int32),
            pltpu.VMEM((2, _L), jnp.int32),
            pltpu.SemaphoreType.DMA,
            pltpu.SemaphoreType.DMA,
        ],
        compiler_params=pltpu.CompilerParams(skip_device_barrier=True),
    )
    def conf(out_hbm, tgt_hbm, res_hbm, o_v, t_v, r_v, sem0, sem1):
        nc = lax.axis_index("c")
        ns = lax.axis_index("s")
        wid = ns * 2 + nc
        row0 = wid * _ROWS_W
        sems = (sem0, sem1)

        def start(buf, c):
            r = row0 + c * _CH_ROWS
            return (
                pltpu.async_copy(
                    out_hbm.at[pl.ds(r, _CH_ROWS), :], o_v.at[buf], sems[buf]
                ),
                pltpu.async_copy(
                    tgt_hbm.at[pl.ds(r, _CH_ROWS), :], t_v.at[buf], sems[buf]
                ),
            )

        zeros = jnp.zeros((_L,), jnp.int32)
        tail_ok = lax.iota(jnp.int32, _L) >= _TAIL_NEW

        handles = [None, None]
        handles[0] = start(0, 0)

        acc_ac = zeros
        acc_b = zeros
        for c in range(_NCH):
            buf = c % _NBUF
            if c + 1 < _NCH:
                handles[(c + 1) % _NBUF] = start((c + 1) % _NBUF, c + 1)
            ha, hb = handles[buf]
            ha.wait()
            hb.wait()

            def body(r, carry, buf=buf):
                a_ac, a_b = carry
                for j in range(_FULL_VECS):
                    o = o_v[buf, r, pl.ds(j * _L, _L)]
                    t = t_v[buf, r, pl.ds(j * _L, _L)]
                    p = o > 0.0
                    a_ac = a_ac + jnp.where(p, t + (1 << 16), zeros)
                    a_b = a_b + t
                o = o_v[buf, r, pl.ds(_TAIL_OFF, _L)]
                t = t_v[buf, r, pl.ds(_TAIL_OFF, _L)]
                p = jnp.logical_and(o > 0.0, tail_ok)
                a_ac = a_ac + jnp.where(p, t + (1 << 16), zeros)
                a_b = a_b + jnp.where(tail_ok, t, zeros)
                return (a_ac, a_b)

            acc_ac, acc_b = lax.fori_loop(
                0, _CH_ROWS, body, (acc_ac, acc_b), unroll=2
            )

        r_v[0, :] = acc_ac
        r_v[1, :] = acc_b
        pltpu.sync_copy(r_v, res_hbm.at[wid])

    return conf


_conf = _make_conf_kernel()


def kernel(output, target):
    res = _conf(output, target)
    a = jnp.sum(res[:, 0, :] >> 16)
    c = jnp.sum(res[:, 0, :] & 0xFFFF)
    b = jnp.sum(res[:, 1, :])
    tp = c
    fp = a - c
    fn = _TOTAL - b - a + c
    return (tp, fp, fn)
